# baseline (speedup 1.0000x reference)
import functools

import numpy as np

import jax
import jax.numpy as jnp
from jax.experimental import pallas as pl
from jax.experimental.pallas import tpu as pltpu

_STREAM_MIN = 4096
_CP = 1024


def _level_perms(L):
    sig = [np.zeros(1, np.int64)]
    for l in range(1, L):
        prev = sig[l - 1]
        sig.append(np.concatenate([2 * prev, 2 * prev + 1]))
    return sig


def _cell(x, w_iou, b_iou, u_iou, hs, fc, H):
    iou = jnp.dot(x, w_iou, preferred_element_type=jnp.float32)
    if hs is not None:
        iou = iou + jnp.dot(hs, u_iou, preferred_element_type=jnp.float32)
    iou = iou + b_iou
    i_g = jax.nn.sigmoid(iou[:, :H])
    o_g = jax.nn.sigmoid(iou[:, H:2 * H])
    u_g = jnp.tanh(iou[:, 2 * H:])
    c = i_g * u_g
    if fc is not None:
        c = c + fc
    h = o_g * jnp.tanh(c)
    return h, c


def _stream_body(args, *, H, leaf):
    if leaf:
        (xl_ref, xr_ref, xp_ref, w_iou_ref, b_iou_ref, u_iou_ref,
         w_f_ref, b_f_ref, u_f_ref, w_cls_ref, b_cls_ref,
         yl_ref, yr_ref, hsum_ref, fc_ref) = args
        hsl = hsr = fcl = fcr = None
    else:
        (xl_ref, xr_ref, xp_ref, hsl_ref, hsr_ref, fcl_ref, fcr_ref,
         w_iou_ref, b_iou_ref, u_iou_ref, w_f_ref, b_f_ref, u_f_ref,
         w_cls_ref, b_cls_ref, yl_ref, yr_ref, hsum_ref, fc_ref) = args
        hsl, hsr = hsl_ref[...], hsr_ref[...]
        fcl, fcr = fcl_ref[...], fcr_ref[...]
    w_iou = w_iou_ref[...]
    b_iou = b_iou_ref[...]
    u_iou = u_iou_ref[...] if not leaf else None
    u_f = u_f_ref[...]
    w_cls = w_cls_ref[...]
    b_cls = b_cls_ref[...]
    hl, cl = _cell(xl_ref[...], w_iou, b_iou, u_iou, hsl, fcl, H)
    hr, cr = _cell(xr_ref[...], w_iou, b_iou, u_iou, hsr, fcr, H)
    hl16 = hl.astype(u_f.dtype)
    hr16 = hr.astype(u_f.dtype)
    yl_ref[...] = jax.nn.sigmoid(
        jnp.dot(hl16, w_cls, preferred_element_type=jnp.float32) + b_cls)
    yr_ref[...] = jax.nn.sigmoid(
        jnp.dot(hr16, w_cls, preferred_element_type=jnp.float32) + b_cls)
    xf = jnp.dot(xp_ref[...], w_f_ref[...],
                 preferred_element_type=jnp.float32) + b_f_ref[...]
    fl = jax.nn.sigmoid(
        xf + jnp.dot(hl16, u_f, preferred_element_type=jnp.float32))
    fr = jax.nn.sigmoid(
        xf + jnp.dot(hr16, u_f, preferred_element_type=jnp.float32))
    hsum_ref[...] = (hl + hr).astype(hsum_ref.dtype)
    fc_ref[...] = fl * cl + fr * cr


def _final_body(feat_ref, hs_in_ref, fci_in_ref, w_iou_ref, b_iou_ref,
                u_iou_ref, w_f_ref, b_f_ref, u_f_ref, w_cls_ref, b_cls_ref,
                y_ref, hsum_ref, fc_ref, xiou_ref, xf_ref, h_all_ref,
                *, l_top, H, top_is_leaf):
    b_iou = b_iou_ref[...]
    u_iou = u_iou_ref[...]
    u_f = u_f_ref[...]
    n_rows = feat_ref.shape[0]
    xiou_ref[...] = jnp.dot(feat_ref[...], w_iou_ref[...],
                            preferred_element_type=jnp.float32) + b_iou
    xf_ref[...] = jnp.dot(feat_ref[:n_rows // 2, :], w_f_ref[...],
                          preferred_element_type=jnp.float32) + b_f_ref[...]
    for l in range(l_top, -1, -1):
        M = 1 << l
        iou = xiou_ref[M:2 * M, :]
        if l == l_top and top_is_leaf:
            hs = fc_in = None
        elif l == l_top:
            hs = hs_in_ref[0:M, :]
            fc_in = fci_in_ref[0:M, :]
        else:
            hs = hsum_ref[0:M, :]
            fc_in = fc_ref[0:M, :]
        if hs is not None:
            iou = iou + jnp.dot(hs, u_iou, preferred_element_type=jnp.float32)
        i_g = jax.nn.sigmoid(iou[:, :H])
        o_g = jax.nn.sigmoid(iou[:, H:2 * H])
        u_g = jnp.tanh(iou[:, 2 * H:])
        c = i_g * u_g
        if fc_in is not None:
            c = c + fc_in
        h = o_g * jnp.tanh(c)
        h16 = h.astype(h_all_ref.dtype)
        h_all_ref[M:2 * M, :] = h16
        if l > 0:
            hp = M // 2
            xf = xf_ref[hp:M, :]
            fl = jax.nn.sigmoid(xf + jnp.dot(
                h16[:hp, :], u_f, preferred_element_type=jnp.float32))
            fr = jax.nn.sigmoid(xf + jnp.dot(
                h16[hp:, :], u_f, preferred_element_type=jnp.float32))
            hsum_ref[0:hp, :] = (h[:hp, :] + h[hp:, :]).astype(hsum_ref.dtype)
            fc_ref[0:hp, :] = fl * c[:hp, :] + fr * c[hp:, :]
    y = jnp.dot(h_all_ref[...], w_cls_ref[...],
                preferred_element_type=jnp.float32)
    y_ref[...] = jax.nn.sigmoid(y + b_cls_ref[...])


def kernel(features, node_evaluation_order, edge_evaluation_order,
           edge_offsets, W_iou, b_iou, U_iou, W_f, b_f, U_f, W_cls, b_cls):
    N, F = features.shape
    H = U_f.shape[0]
    L = (N + 1).bit_length() - 1
    sig = _level_perms(L)

    bf16 = jnp.bfloat16
    gather_idx = np.zeros(N + 1, np.int32)
    for l in range(L):
        gather_idx[(1 << l):(2 << l)] = (1 << l) + sig[l]
    featp0 = jnp.concatenate(
        [jnp.zeros((1, F), bf16), features.astype(bf16)], axis=0)
    featp = jnp.take(featp0, jnp.asarray(gather_idx), axis=0)

    b_iou2 = b_iou.reshape(1, -1).astype(jnp.float32)
    b_f2 = b_f.reshape(1, -1).astype(jnp.float32)
    b_cls2 = b_cls.reshape(1, -1).astype(jnp.float32)
    weights = (W_iou.astype(bf16), b_iou2, U_iou.astype(bf16),
               W_f.astype(bf16), b_f2, U_f.astype(bf16),
               W_cls.astype(bf16), b_cls2)
    wspecs = [pl.BlockSpec(w.shape, lambda i: (0, 0)) for w in weights]

    stream_levels = [l for l in range(L - 1, -1, -1)
                     if (1 << l) >= max(_STREAM_MIN, 4 * _CP)]
    hsum = fc = None
    ys = []
    for l in stream_levels:
        M = 1 << l
        hp = M // 2
        C = _CP
        nb = hp // C
        leaf = l == L - 1
        xl_spec = pl.BlockSpec((C, F), lambda i, b=M // C: (b + i, 0))
        xr_spec = pl.BlockSpec(
            (C, F), lambda i, b=(M + hp) // C: (b + i, 0))
        xp_spec = pl.BlockSpec((C, F), lambda i, b=hp // C: (b + i, 0))
        operands = [featp, featp, featp]
        in_specs = [xl_spec, xr_spec, xp_spec]
        if not leaf:
            operands += [hsum, hsum, fc, fc]
            in_specs += [
                pl.BlockSpec((C, H), lambda i: (i, 0)),
                pl.BlockSpec((C, H), lambda i, b=nb: (b + i, 0)),
                pl.BlockSpec((C, H), lambda i: (i, 0)),
                pl.BlockSpec((C, H), lambda i, b=nb: (b + i, 0)),
            ]
        operands += list(weights)
        in_specs += wspecs
        body = functools.partial(
            lambda *args, H, leaf: _stream_body(args, H=H, leaf=leaf),
            H=H, leaf=leaf)
        yl, yr, hsum, fc = pl.pallas_call(
            body,
            grid=(nb,),
            in_specs=in_specs,
            out_specs=[
                pl.BlockSpec((C, 1), lambda i: (i, 0)),
                pl.BlockSpec((C, 1), lambda i: (i, 0)),
                pl.BlockSpec((C, H), lambda i: (i, 0)),
                pl.BlockSpec((C, H), lambda i: (i, 0)),
            ],
            out_shape=[
                jax.ShapeDtypeStruct((hp, 1), jnp.float32),
                jax.ShapeDtypeStruct((hp, 1), jnp.float32),
                jax.ShapeDtypeStruct((hp, H), jnp.bfloat16),
                jax.ShapeDtypeStruct((hp, H), jnp.float32),
            ],
        )(*operands)
        ys.append(jnp.concatenate([yl, yr], axis=0))

    l_top = (stream_levels[-1] - 1) if stream_levels else L - 1
    top_is_leaf = not stream_levels
    M_top = 1 << l_top
    feat_small = featp[:2 * M_top]
    if top_is_leaf:
        hsum = jnp.zeros((max(8, M_top), H), jnp.bfloat16)
        fc = jnp.zeros((max(8, M_top), H), jnp.float32)
    scratch_rows = max(8, M_top // 2)
    body = functools.partial(_final_body, l_top=l_top, H=H,
                             top_is_leaf=top_is_leaf)
    y_small = pl.pallas_call(
        body,
        out_shape=jax.ShapeDtypeStruct((2 * M_top, 1), jnp.float32),
        scratch_shapes=[
            pltpu.VMEM((scratch_rows, H), jnp.bfloat16),
            pltpu.VMEM((scratch_rows, H), jnp.float32),
            pltpu.VMEM((2 * M_top, 3 * H), jnp.float32),
            pltpu.VMEM((M_top, H), jnp.float32),
            pltpu.VMEM((2 * M_top, H), jnp.bfloat16),
        ],
    )(feat_small, hsum, fc, *weights)

    parts = [y_small[1:]] + [ys[i] for i in range(len(ys) - 1, -1, -1)]
    ids = np.zeros(N, np.int64)
    for l in range(l_top + 1):
        ids[(1 << l) - 1:(2 << l) - 1] = (1 << l) - 1 + sig[l]
    ofs = 2 * M_top - 1
    for l in sorted(stream_levels):
        ids[ofs:ofs + (1 << l)] = (1 << l) - 1 + sig[l]
        ofs += 1 << l
    unperm = np.zeros(N, np.int32)
    unperm[ids] = np.arange(N, dtype=np.int32)
    return jnp.take(jnp.concatenate(parts, axis=0),
                    jnp.asarray(unperm), axis=0)


# speedup vs baseline: 3.1968x; 3.1968x over previous
import functools

import jax
import jax.numpy as jnp
from jax.experimental import pallas as pl
from jax.experimental.pallas import tpu as pltpu

_C = 2048


def _body(feat_ref, w_iou_ref, b_iou_ref, u_iou_ref, w_f_ref, b_f_ref,
          u_f_ref, w_cls_ref, b_cls_ref, y_ref, hsum_ref, fc_ref, *, L, H):
    w_iou = w_iou_ref[...]
    b_iou = b_iou_ref[...]
    u_iou = u_iou_ref[...]
    w_f = w_f_ref[...]
    b_f = b_f_ref[...]
    u_f = u_f_ref[...]
    w_cls = w_cls_ref[...]
    b_cls = b_cls_ref[...]
    for l in range(L - 1, -1, -1):
        M = 1 << l
        cs = min(M, _C)
        for i in range(M // cs):
            r0 = M + i * cs
            x = feat_ref[r0:r0 + cs, :]
            iou = jnp.dot(x, w_iou, preferred_element_type=jnp.float32)
            if l < L - 1:
                hs = hsum_ref[i * cs:(i + 1) * cs, :]
                iou = iou + jnp.dot(hs, u_iou,
                                    preferred_element_type=jnp.float32)
            iou = iou + b_iou
            i_g = jax.nn.sigmoid(iou[:, :H])
            o_g = jax.nn.sigmoid(iou[:, H:2 * H])
            u_g = jnp.tanh(iou[:, 2 * H:])
            c_l = i_g * u_g
            if l < L - 1:
                c_l = c_l + fc_ref[i * cs:(i + 1) * cs, :]
            h_l = o_g * jnp.tanh(c_l)
            h16 = h_l.astype(jnp.bfloat16)
            y = jax.lax.dot_general(
                w_cls, h16, (((1,), (1,)), ((), ())),
                preferred_element_type=jnp.float32)
            y_ref[0:1, r0:r0 + cs] = jax.nn.sigmoid(y + b_cls)
            if l > 0:
                hp = cs // 2
                p0 = M // 2 + i * hp
                xp = feat_ref[p0:p0 + hp, :]
                xf = jnp.dot(xp, w_f, preferred_element_type=jnp.float32)
                xf = xf + b_f
                xrep = jnp.broadcast_to(
                    xf[:, None, :], (hp, 2, H)).reshape(cs, H)
                f = jax.nn.sigmoid(
                    xrep + jnp.dot(h16, u_f,
                                   preferred_element_type=jnp.float32))
                fc2 = f * c_l
                fc_ref[i * hp:(i + 1) * hp, :] = (
                    fc2.reshape(hp, 2, H).sum(axis=1))
                hsum_ref[i * hp:(i + 1) * hp, :] = (
                    h_l.reshape(hp, 2, H).sum(axis=1).astype(hsum_ref.dtype))


def kernel(features, node_evaluation_order, edge_evaluation_order,
           edge_offsets, W_iou, b_iou, U_iou, W_f, b_f, U_f, W_cls, b_cls):
    N, F = features.shape
    H = U_f.shape[0]
    L = (N + 1).bit_length() - 1

    bf16 = jnp.bfloat16
    featp = jnp.concatenate(
        [jnp.zeros((1, F), bf16), features.astype(bf16)], axis=0)
    weights = (W_iou.astype(bf16), b_iou.reshape(1, -1).astype(jnp.float32),
               U_iou.astype(bf16), W_f.astype(bf16),
               b_f.reshape(1, -1).astype(jnp.float32), U_f.astype(bf16),
               W_cls.reshape(1, -1).astype(bf16),
               b_cls.reshape(1, 1).astype(jnp.float32))

    half = max(8, (N + 1) // 4)
    body = functools.partial(_body, L=L, H=H)
    y = pl.pallas_call(
        body,
        out_shape=jax.ShapeDtypeStruct((1, N + 1), jnp.float32),
        scratch_shapes=[
            pltpu.VMEM((half, H), jnp.bfloat16),
            pltpu.VMEM((half, H), jnp.float32),
        ],
    )(featp, *weights)
    return y.reshape(N + 1, 1)[1:]


# speedup vs baseline: 3.1990x; 1.0007x over previous
import functools

import jax
import jax.numpy as jnp
from jax.experimental import pallas as pl
from jax.experimental.pallas import tpu as pltpu

_C = 2048


def _sig(x):
    return 0.5 * jnp.tanh(0.5 * x) + 0.5


def _body(feat_ref, w_iou_ref, b_iou_ref, u_iou_ref, w_f_ref, b_f_ref,
          u_f_ref, w_cls_ref, b_cls_ref, y_ref, hsum_ref, fc_ref, *, L, H):
    w_iou = w_iou_ref[...]
    b_iou = b_iou_ref[...]
    u_iou = u_iou_ref[...]
    w_f = w_f_ref[...]
    b_f = b_f_ref[...]
    u_f = u_f_ref[...]
    w_cls = w_cls_ref[...]
    b_cls = b_cls_ref[...]
    for l in range(L - 1, -1, -1):
        M = 1 << l
        cs = min(M, _C)
        for i in range(M // cs):
            r0 = M + i * cs
            x = feat_ref[r0:r0 + cs, :]
            iou = jnp.dot(x, w_iou, preferred_element_type=jnp.float32)
            if l < L - 1:
                hs = hsum_ref[i * cs:(i + 1) * cs, :]
                iou = iou + jnp.dot(hs, u_iou,
                                    preferred_element_type=jnp.float32)
            iou = iou + b_iou
            i_g = _sig(iou[:, :H])
            o_g = _sig(iou[:, H:2 * H])
            u_g = jnp.tanh(iou[:, 2 * H:])
            c_l = i_g * u_g
            if l < L - 1:
                c_l = c_l + fc_ref[i * cs:(i + 1) * cs, :]
            h_l = o_g * jnp.tanh(c_l)
            h16 = h_l.astype(jnp.bfloat16)
            y = jax.lax.dot_general(
                w_cls, h16, (((1,), (1,)), ((), ())),
                preferred_element_type=jnp.float32)
            y_ref[0:1, r0:r0 + cs] = _sig(y + b_cls)
            if l > 0:
                hp = cs // 2
                p0 = M // 2 + i * hp
                xp = feat_ref[p0:p0 + hp, :]
                xf = jnp.dot(xp, w_f, preferred_element_type=jnp.float32)
                xf = xf + b_f
                xrep = jnp.broadcast_to(
                    xf[:, None, :], (hp, 2, H)).reshape(cs, H)
                f = _sig(
                    xrep + jnp.dot(h16, u_f,
                                   preferred_element_type=jnp.float32))
                fc2 = f * c_l
                fc_ref[i * hp:(i + 1) * hp, :] = (
                    fc2.reshape(hp, 2, H).sum(axis=1))
                hsum_ref[i * hp:(i + 1) * hp, :] = (
                    h_l.reshape(hp, 2, H).sum(axis=1).astype(hsum_ref.dtype))


def kernel(features, node_evaluation_order, edge_evaluation_order,
           edge_offsets, W_iou, b_iou, U_iou, W_f, b_f, U_f, W_cls, b_cls):
    N, F = features.shape
    H = U_f.shape[0]
    L = (N + 1).bit_length() - 1

    bf16 = jnp.bfloat16
    featp = jnp.concatenate(
        [jnp.zeros((1, F), bf16), features.astype(bf16)], axis=0)
    weights = (W_iou.astype(bf16), b_iou.reshape(1, -1).astype(jnp.float32),
               U_iou.astype(bf16), W_f.astype(bf16),
               b_f.reshape(1, -1).astype(jnp.float32), U_f.astype(bf16),
               W_cls.reshape(1, -1).astype(bf16),
               b_cls.reshape(1, 1).astype(jnp.float32))

    half = max(8, (N + 1) // 4)
    body = functools.partial(_body, L=L, H=H)
    y = pl.pallas_call(
        body,
        out_shape=jax.ShapeDtypeStruct((1, N + 1), jnp.float32),
        scratch_shapes=[
            pltpu.VMEM((half, H), jnp.bfloat16),
            pltpu.VMEM((half, H), jnp.float32),
        ],
    )(featp, *weights)
    return y.reshape(N + 1, 1)[1:]
